# Initial kernel scaffold; baseline (speedup 1.0000x reference)
import jax
import jax.numpy as jnp
from jax.experimental import pallas as pl


def kernel(x, edge_index, edge_weight, W):
    raise NotImplementedError("write your pallas kernel here")


# speedup vs baseline: 3.3210x; 3.3210x over previous
import functools

import jax
import jax.numpy as jnp
from jax import lax
from jax.experimental import pallas as pl
from jax.experimental.pallas import tpu as pltpu
from jax.experimental.pallas import tpu_sc as plsc

NC = 2
NS = 16
L = 16
C = 128


def _sc_segment_sum(N, D, E_pad, n_pad):
    per_w = E_pad // (NC * NS)
    n_chunks = per_w // C
    rpt = n_pad // NS
    zcopies = rpt // C

    mesh = plsc.VectorSubcoreMesh(core_axis_name="c", subcore_axis_name="s")

    @functools.partial(
        pl.kernel,
        out_type=jax.ShapeDtypeStruct((NC, n_pad, D), jnp.float32),
        mesh=mesh,
        scratch_types=[
            pltpu.VMEM((C,), jnp.int32),
            pltpu.VMEM((C,), jnp.int32),
            pltpu.VMEM((C,), jnp.float32),
            pltpu.VMEM((C, D), jnp.float32),
            pltpu.VMEM_SHARED((n_pad, D), jnp.float32),
            pltpu.SemaphoreType.DMA,
        ],
    )
    def body(x_hbm, src_hbm, dst_hbm, w_hbm, out_hbm, srcv, dstv, wv, rows,
             acc, sem):
        cid = lax.axis_index("c")
        sid = lax.axis_index("s")
        wid = cid * NS + sid

        zero = jnp.zeros((L,), jnp.float32)

        def zrow(i, carry):
            for j in range(D // L):
                rows[i, pl.ds(j * L, L)] = zero
            return carry

        lax.fori_loop(0, C, zrow, 0)
        for t in range(zcopies):
            pltpu.sync_copy(rows, acc.at[pl.ds(sid * rpt + t * C, C)])
        plsc.subcore_barrier()

        base = wid * per_w

        def chunk(k, carry):
            off = base + k * C
            pltpu.sync_copy(src_hbm.at[pl.ds(off, C)], srcv)
            pltpu.sync_copy(dst_hbm.at[pl.ds(off, C)], dstv)
            pltpu.sync_copy(w_hbm.at[pl.ds(off, C)], wv)
            pltpu.async_copy(x_hbm.at[srcv], rows, sem).wait()

            def egroup(g, c2):
                wvec = wv[pl.ds(g * L, L)]
                for i in range(L):
                    e = g * L + i
                    we = wvec[i]
                    for j in range(D // L):
                        sl = pl.ds(j * L, L)
                        rows[e, sl] = rows[e, sl] * we
                return c2

            lax.fori_loop(0, C // L, egroup, 0)
            pltpu.sync_copy(rows, acc.at[dstv], add=True)
            return carry

        lax.fori_loop(0, n_chunks, chunk, 0)
        plsc.subcore_barrier()

        pltpu.sync_copy(acc.at[pl.ds(sid * rpt, rpt)],
                        out_hbm.at[cid, pl.ds(sid * rpt, rpt)])

    return body


def _tc_finish(N, D, n_pad, BR):
    def body(p_ref, w_ref, o_ref):
        p = p_ref[0] + p_ref[1]
        o_ref[...] = jnp.tanh(
            lax.dot_general(p, w_ref[...], (((1,), (0,)), ((), ())),
                            preferred_element_type=jnp.float32))

    return pl.pallas_call(
        body,
        grid=(N // BR,),
        in_specs=[
            pl.BlockSpec((2, BR, D), lambda i: (0, i, 0)),
            pl.BlockSpec((D, D), lambda i: (0, 0)),
        ],
        out_specs=pl.BlockSpec((BR, D), lambda i: (i, 0)),
        out_shape=jax.ShapeDtypeStruct((N, D), jnp.float32),
    )


def kernel(x, edge_index, edge_weight, W):
    N, D = x.shape
    E = edge_index.shape[1]

    src = edge_index[0].astype(jnp.int32)
    dst = edge_index[1].astype(jnp.int32)
    w = edge_weight.astype(jnp.float32)

    quantum = NC * NS * C
    E_pad = ((E + quantum - 1) // quantum) * quantum
    pad = E_pad - E
    if pad:
        src = jnp.concatenate([src, jnp.zeros((pad,), jnp.int32)])
        dst = jnp.concatenate([dst, jnp.zeros((pad,), jnp.int32)])
        w = jnp.concatenate([w, jnp.zeros((pad,), jnp.float32)])

    n_quantum = NS * C
    n_pad = ((N + n_quantum - 1) // n_quantum) * n_quantum

    partials = _sc_segment_sum(N, D, E_pad, n_pad)(x, src, dst, w)

    BR = 400
    assert N % BR == 0
    out = _tc_finish(N, D, n_pad, BR)(partials[:, :N], W)
    return out
